# baseline (speedup 1.0000x reference)
import jax
import jax.numpy as jnp
from jax.experimental import pallas as pl
from jax.experimental.pallas import tpu as pltpu

_N = 10000
_D = 128
_CHUNK = 40
_NBUF = 16
_NCHUNK = _N // _CHUNK


def _gcn_kernel(a_hbm, x_ref, w_ref, b_ref, out_ref, buf, s_ref, sems):
    def _copy(c, slot):
        return pltpu.make_async_copy(
            a_hbm.at[pl.ds(c * _CHUNK, _CHUNK), :],
            buf.at[slot],
            sems.at[slot],
        )

    for k in range(_NBUF):
        _copy(k, k).start()

    s_ref[...] = jnp.dot(x_ref[...], w_ref[...],
                         preferred_element_type=jnp.float32,
                         precision=jax.lax.Precision.DEFAULT)

    def _step(c, carry):
        slot = jax.lax.rem(c, _NBUF)
        _copy(c, slot).wait()
        out_ref[pl.ds(c * _CHUNK, _CHUNK), :] = (
            jnp.dot(buf[slot], s_ref[...],
                    preferred_element_type=jnp.float32,
                    precision=jax.lax.Precision.DEFAULT)
            + b_ref[...]
        )

        @pl.when(c + _NBUF < _NCHUNK)
        def _():
            _copy(c + _NBUF, slot).start()

        return carry

    jax.lax.fori_loop(0, _NCHUNK, _step, 0)


def kernel(x, edge_index, weight, bias):
    return pl.pallas_call(
        _gcn_kernel,
        in_specs=[
            pl.BlockSpec(memory_space=pltpu.MemorySpace.HBM),
            pl.BlockSpec(memory_space=pltpu.MemorySpace.VMEM),
            pl.BlockSpec(memory_space=pltpu.MemorySpace.VMEM),
            pl.BlockSpec(memory_space=pltpu.MemorySpace.VMEM),
        ],
        out_specs=pl.BlockSpec(memory_space=pltpu.MemorySpace.VMEM),
        out_shape=jax.ShapeDtypeStruct((_N, _D), jnp.float32),
        scratch_shapes=[
            pltpu.VMEM((_NBUF, _CHUNK, _N), jnp.float32),
            pltpu.VMEM((_N, _D), jnp.float32),
            pltpu.SemaphoreType.DMA((_NBUF,)),
        ],
    )(edge_index, x, weight, bias.reshape(1, _D))


# speedup vs baseline: 1.1898x; 1.1898x over previous
import jax
import jax.numpy as jnp
from jax.experimental import pallas as pl
from jax.experimental.pallas import tpu as pltpu

_N = 10000
_D = 128
_BM = 400


def _gcn_kernel(a_ref, x_ref, w_ref, b_ref, out_ref, s_ref):
    @pl.when(pl.program_id(0) == 0)
    def _():
        xb = x_ref[...].astype(jnp.bfloat16)
        wb = w_ref[...].astype(jnp.bfloat16)
        s_ref[...] = jnp.dot(xb, wb, preferred_element_type=jnp.float32
                             ).astype(jnp.bfloat16)

    a = a_ref[...].astype(jnp.bfloat16)
    out_ref[...] = (
        jnp.dot(a, s_ref[...], preferred_element_type=jnp.float32)
        + b_ref[...]
    )


def kernel(x, edge_index, weight, bias):
    return pl.pallas_call(
        _gcn_kernel,
        grid=(_N // _BM,),
        in_specs=[
            pl.BlockSpec((_BM, _N), lambda i: (i, 0)),
            pl.BlockSpec((_N, _D), lambda i: (0, 0)),
            pl.BlockSpec((_D, _D), lambda i: (0, 0)),
            pl.BlockSpec((1, _D), lambda i: (0, 0)),
        ],
        out_specs=pl.BlockSpec((_BM, _D), lambda i: (i, 0)),
        out_shape=jax.ShapeDtypeStruct((_N, _D), jnp.float32),
        scratch_shapes=[pltpu.VMEM((_N, _D), jnp.bfloat16)],
        compiler_params=pltpu.CompilerParams(
            dimension_semantics=("arbitrary",),
        ),
    )(edge_index, x, weight, bias.reshape(1, _D))
